# baseline (speedup 1.0000x reference)
import functools

import jax
import jax.numpy as jnp
from jax import lax
from jax.experimental import pallas as pl
from jax.experimental.pallas import tpu as pltpu
from jax.experimental.pallas import tpu_sc as plsc

_N_EMB = 1024
_EMB_DIM = 256
_BETA = 0.25
_B = 8
_P = 1024

_NC = 2
_NS = 16
_NW = _NC * _NS
_CPW = _EMB_DIM // _NW


def _argmin_body(xb_ref, e_ref, esq_ref, idx_ref, loss_ref):
    xb = xb_ref[0]
    emb_tab = e_ref[...]
    esq = esq_ref[...]

    s = jax.lax.dot_general(
        emb_tab, xb, (((1,), (0,)), ((), ())),
        preferred_element_type=jnp.float32)
    xsq = jnp.sum(xb * xb, axis=0, keepdims=True)
    dist = (xsq + esq) - 2.0 * s

    mind = jnp.min(dist, axis=0, keepdims=True)
    iota = jax.lax.broadcasted_iota(jnp.int32, dist.shape, 0)
    idx_ref[0] = jnp.min(jnp.where(dist == mind, iota, _N_EMB),
                         axis=0, keepdims=True)

    @pl.when(pl.program_id(0) == 0)
    def _init():
        loss_ref[...] = jnp.zeros_like(loss_ref)

    loss_ref[...] += jnp.sum(mind, axis=(0, 1), keepdims=True)


@jax.jit
def _vq_argmin(xr, embeddings, esq):
    return pl.pallas_call(
        _argmin_body,
        grid=(_B,),
        in_specs=[
            pl.BlockSpec((1, _EMB_DIM, _P), lambda i: (i, 0, 0)),
            pl.BlockSpec((_N_EMB, _EMB_DIM), lambda i: (0, 0)),
            pl.BlockSpec((_N_EMB, 1), lambda i: (0, 0)),
        ],
        out_specs=[
            pl.BlockSpec((1, 1, _P), lambda i: (i, 0, 0)),
            pl.BlockSpec((1, 1), lambda i: (0, 0)),
        ],
        out_shape=[
            jax.ShapeDtypeStruct((_B, 1, _P), jnp.int32),
            jax.ShapeDtypeStruct((1, 1), jnp.float32),
        ],
    )(xr, embeddings, esq)


def _gather_body(et_hbm, idx_hbm, out_hbm, et_v, idx_v, out_v, sems):
    wid = lax.axis_index("s") * _NC + lax.axis_index("c")
    c0 = wid * _CPW
    pltpu.sync_copy(et_hbm.at[pl.ds(c0 * _N_EMB, _CPW * _N_EMB)], et_v)
    pltpu.sync_copy(idx_hbm, idx_v)

    copies = []
    for b in range(_B):
        slot = b % 2
        if b >= 2:
            copies[b - 2].wait()

        def step(j, carry, b=b, slot=slot):
            idxv = idx_v[b, pl.ds(j * 16, 16)]
            for c in range(_CPW):
                out_v[slot, c, pl.ds(j * 16, 16)] = plsc.load_gather(
                    et_v, [idxv + jnp.int32(c * _N_EMB)])
            return carry

        lax.fori_loop(0, _P // 16, step, 0, unroll=4)
        cp = pltpu.make_async_copy(
            out_v.at[slot], out_hbm.at[pl.ds(b * _EMB_DIM + c0, _CPW)],
            sems.at[slot])
        cp.start()
        copies.append(cp)
    copies[-2].wait()
    copies[-1].wait()


@jax.jit
def _vq_gather(et, idx):
    return pl.kernel(
        _gather_body,
        out_type=jax.ShapeDtypeStruct((_B * _EMB_DIM, _P), jnp.float32),
        mesh=plsc.VectorSubcoreMesh(core_axis_name="c", subcore_axis_name="s",
                                    num_cores=_NC, num_subcores=_NS),
        compiler_params=pltpu.CompilerParams(needs_layout_passes=False),
        scratch_types=[
            pltpu.VMEM((_CPW * _N_EMB,), jnp.float32),
            pltpu.VMEM((_B, _P), jnp.int32),
            pltpu.VMEM((2, _CPW, _P), jnp.float32),
            pltpu.SemaphoreType.DMA((2,)),
        ],
    )(et, idx)


def kernel(x, embeddings):
    b, c, h, w = x.shape
    xr = x.reshape(b, c, h * w)
    esq = jnp.sum(embeddings ** 2, axis=-1)[:, None]
    idx, loss_sum = _vq_argmin(xr, embeddings, esq)
    emb_r = _vq_gather(embeddings.T.reshape(-1), idx.reshape(b, h * w))
    emb = emb_r.reshape(b, c, h, w)
    loss = loss_sum[0, 0] * ((1.0 + _BETA) / (b * c * h * w))
    return emb, loss


# speedup vs baseline: 2.0945x; 2.0945x over previous
import functools

import jax
import jax.numpy as jnp
from jax import lax
from jax.experimental import pallas as pl
from jax.experimental.pallas import tpu as pltpu
from jax.experimental.pallas import tpu_sc as plsc

_N_EMB = 1024
_EMB_DIM = 256
_BETA = 0.25
_N = 8192
_BLK = 1024

_NC = 2
_NS = 16
_NW = _NC * _NS
_RPW = _N // _NW


def _argmin_body(fx_ref, e_ref, esq_ref, idx_ref, loss_ref):
    fx = fx_ref[...]
    emb_tab = e_ref[...]
    esq = esq_ref[...]

    s = jax.lax.dot_general(
        fx, emb_tab, (((1,), (1,)), ((), ())),
        preferred_element_type=jnp.float32)
    xsq = jnp.sum(fx * fx, axis=1, keepdims=True)
    dist = (xsq + esq) - 2.0 * s

    mind = jnp.min(dist, axis=1, keepdims=True)
    iota = jax.lax.broadcasted_iota(jnp.int32, dist.shape, 1)
    idx_ref[...] = jnp.min(jnp.where(dist == mind, iota, _N_EMB),
                           axis=1, keepdims=True)

    @pl.when(pl.program_id(0) == 0)
    def _init():
        loss_ref[...] = jnp.zeros_like(loss_ref)

    loss_ref[...] += jnp.sum(mind, axis=(0, 1), keepdims=True)


@jax.jit
def _vq_argmin(fx, embeddings, esq):
    return pl.pallas_call(
        _argmin_body,
        grid=(_N // _BLK,),
        in_specs=[
            pl.BlockSpec((_BLK, _EMB_DIM), lambda i: (i, 0)),
            pl.BlockSpec((_N_EMB, _EMB_DIM), lambda i: (0, 0)),
            pl.BlockSpec((1, _N_EMB), lambda i: (0, 0)),
        ],
        out_specs=[
            pl.BlockSpec((_BLK, 1), lambda i: (i, 0)),
            pl.BlockSpec((1, 1), lambda i: (0, 0)),
        ],
        out_shape=[
            jax.ShapeDtypeStruct((_N, 1), jnp.int32),
            jax.ShapeDtypeStruct((1, 1), jnp.float32),
        ],
    )(fx, embeddings, esq)


def _gather_body(table_hbm, idx_hbm, out_hbm, idx_v, rows_v, sem):
    wid = lax.axis_index("s") * _NC + lax.axis_index("c")
    base = wid * _RPW
    pltpu.sync_copy(idx_hbm.at[pl.ds(base, _RPW)], idx_v)
    pltpu.async_copy(table_hbm.at[idx_v], rows_v, sem).wait()
    pltpu.sync_copy(rows_v, out_hbm.at[pl.ds(base, _RPW)])


@jax.jit
def _vq_gather(table, idx):
    return pl.kernel(
        _gather_body,
        out_type=jax.ShapeDtypeStruct((_N, _EMB_DIM), jnp.float32),
        mesh=plsc.VectorSubcoreMesh(core_axis_name="c", subcore_axis_name="s",
                                    num_cores=_NC, num_subcores=_NS),
        compiler_params=pltpu.CompilerParams(needs_layout_passes=False),
        scratch_types=[
            pltpu.VMEM((_RPW,), jnp.int32),
            pltpu.VMEM((_RPW, _EMB_DIM), jnp.float32),
            pltpu.SemaphoreType.DMA,
        ],
    )(table, idx)


def kernel(x, embeddings):
    b, c, h, w = x.shape
    fx = jnp.transpose(x, (0, 2, 3, 1)).reshape(b * h * w, c)
    esq = jnp.sum(embeddings ** 2, axis=-1)[None, :]
    idx, loss_sum = _vq_argmin(fx, embeddings, esq)
    emb_flat = _vq_gather(embeddings, idx.reshape(-1))
    emb = jnp.transpose(emb_flat.reshape(b, h, w, c), (0, 3, 1, 2))
    loss = loss_sum[0, 0] * ((1.0 + _BETA) / (b * c * h * w))
    return emb, loss


# speedup vs baseline: 2.2360x; 1.0676x over previous
import functools

import jax
import jax.numpy as jnp
from jax import lax
from jax.experimental import pallas as pl
from jax.experimental.pallas import tpu as pltpu
from jax.experimental.pallas import tpu_sc as plsc

_N_EMB = 1024
_EMB_DIM = 256
_BETA = 0.25
_N = 8192
_BLK = 1024

_NC = 2
_NS = 16
_NW = _NC * _NS
_RPW = _N // _NW


def _argmin_body(fx_ref, e_ref, esq_ref, idx_ref, loss_ref):
    xb = jnp.transpose(fx_ref[...], (1, 0))
    emb_tab = e_ref[...]
    esq = esq_ref[...]

    s = jax.lax.dot_general(
        emb_tab, xb, (((1,), (0,)), ((), ())),
        preferred_element_type=jnp.float32)
    xsq = jnp.sum(xb * xb, axis=0, keepdims=True)
    dist = (xsq + esq) - 2.0 * s

    mind = jnp.min(dist, axis=0, keepdims=True)
    iota = jax.lax.broadcasted_iota(jnp.int32, dist.shape, 0)
    idx_ref[0] = jnp.min(jnp.where(dist == mind, iota, _N_EMB),
                         axis=0, keepdims=True)

    @pl.when(pl.program_id(0) == 0)
    def _init():
        loss_ref[...] = jnp.zeros_like(loss_ref)

    loss_ref[...] += jnp.sum(mind, axis=(0, 1), keepdims=True)


@jax.jit
def _vq_argmin(fx, embeddings, esq):
    return pl.pallas_call(
        _argmin_body,
        grid=(_N // _BLK,),
        in_specs=[
            pl.BlockSpec((_BLK, _EMB_DIM), lambda i: (i, 0)),
            pl.BlockSpec((_N_EMB, _EMB_DIM), lambda i: (0, 0)),
            pl.BlockSpec((_N_EMB, 1), lambda i: (0, 0)),
        ],
        out_specs=[
            pl.BlockSpec((1, 1, _BLK), lambda i: (i, 0, 0)),
            pl.BlockSpec((1, 1), lambda i: (0, 0)),
        ],
        out_shape=[
            jax.ShapeDtypeStruct((_N // _BLK, 1, _BLK), jnp.int32),
            jax.ShapeDtypeStruct((1, 1), jnp.float32),
        ],
    )(fx, embeddings, esq)


def _gather_body(table_hbm, idx_hbm, out_hbm, idx_v, rows_v, sem):
    wid = lax.axis_index("s") * _NC + lax.axis_index("c")
    base = wid * _RPW
    pltpu.sync_copy(idx_hbm.at[pl.ds(base, _RPW)], idx_v)
    pltpu.async_copy(table_hbm.at[idx_v], rows_v, sem).wait()
    pltpu.sync_copy(rows_v, out_hbm.at[pl.ds(base, _RPW)])


@jax.jit
def _vq_gather(table, idx):
    return pl.kernel(
        _gather_body,
        out_type=jax.ShapeDtypeStruct((_N, _EMB_DIM), jnp.float32),
        mesh=plsc.VectorSubcoreMesh(core_axis_name="c", subcore_axis_name="s",
                                    num_cores=_NC, num_subcores=_NS),
        compiler_params=pltpu.CompilerParams(needs_layout_passes=False),
        scratch_types=[
            pltpu.VMEM((_RPW,), jnp.int32),
            pltpu.VMEM((_RPW, _EMB_DIM), jnp.float32),
            pltpu.SemaphoreType.DMA,
        ],
    )(table, idx)


def kernel(x, embeddings):
    b, c, h, w = x.shape
    fx = jnp.transpose(x, (0, 2, 3, 1)).reshape(b * h * w, c)
    esq = jnp.sum(embeddings ** 2, axis=-1)[:, None]
    idx, loss_sum = _vq_argmin(fx, embeddings, esq)
    emb_flat = _vq_gather(embeddings, idx.reshape(-1))
    emb = jnp.transpose(emb_flat.reshape(b, h, w, c), (0, 3, 1, 2))
    loss = loss_sum[0, 0] * ((1.0 + _BETA) / (b * c * h * w))
    return emb, loss


# speedup vs baseline: 2.2364x; 1.0002x over previous
import functools

import jax
import jax.numpy as jnp
from jax import lax
from jax.experimental import pallas as pl
from jax.experimental.pallas import tpu as pltpu
from jax.experimental.pallas import tpu_sc as plsc

_N_EMB = 1024
_EMB_DIM = 256
_BETA = 0.25
_N = 8192
_BLK = 1024

_NC = 2
_NS = 16
_NW = _NC * _NS
_RPW = _N // _NW


def _argmin_body(fx_ref, e_ref, esq_ref, idx_ref, loss_ref):
    xb = jnp.transpose(fx_ref[...], (1, 0))
    emb_tab = e_ref[...]
    esq = esq_ref[...]

    s = jax.lax.dot_general(
        emb_tab, xb, (((1,), (0,)), ((), ())),
        preferred_element_type=jnp.float32)
    xsq = jnp.sum(xb * xb, axis=0, keepdims=True)
    dist = (xsq + esq) - 2.0 * s

    mind = jnp.min(dist, axis=0, keepdims=True)
    iota = jax.lax.broadcasted_iota(jnp.int32, dist.shape, 0)
    idx_ref[0] = jnp.min(jnp.where(dist == mind, iota, _N_EMB),
                         axis=0, keepdims=True)

    @pl.when(pl.program_id(0) == 0)
    def _init():
        loss_ref[...] = jnp.zeros_like(loss_ref)

    loss_ref[...] += jnp.sum(mind, axis=(0, 1), keepdims=True)


@jax.jit
def _vq_argmin(fx, embeddings, esq):
    return pl.pallas_call(
        _argmin_body,
        grid=(_N // _BLK,),
        in_specs=[
            pl.BlockSpec((_BLK, _EMB_DIM), lambda i: (i, 0)),
            pl.BlockSpec((_N_EMB, _EMB_DIM), lambda i: (0, 0)),
            pl.BlockSpec((_N_EMB, 1), lambda i: (0, 0)),
        ],
        out_specs=[
            pl.BlockSpec((1, 1, _BLK), lambda i: (i, 0, 0)),
            pl.BlockSpec((1, 1), lambda i: (0, 0)),
        ],
        out_shape=[
            jax.ShapeDtypeStruct((_N // _BLK, 1, _BLK), jnp.int32),
            jax.ShapeDtypeStruct((1, 1), jnp.float32),
        ],
    )(fx, embeddings, esq)


def _gather_body(table_hbm, idx_hbm, out_hbm, idx_v, rows_v, sem):
    wid = lax.axis_index("s") * _NC + lax.axis_index("c")
    base = wid * _RPW
    pltpu.sync_copy(idx_hbm.at[pl.ds(base, _RPW)], idx_v)
    pltpu.async_copy(table_hbm.at[idx_v], rows_v, sem).wait()
    pltpu.sync_copy(rows_v, out_hbm.at[pl.ds(base, _RPW)])


@jax.jit
def _vq_gather(table, idx):
    return pl.kernel(
        _gather_body,
        out_type=jax.ShapeDtypeStruct((_N, _EMB_DIM), jnp.float32),
        mesh=plsc.VectorSubcoreMesh(core_axis_name="c", subcore_axis_name="s",
                                    num_cores=_NC, num_subcores=_NS),
        compiler_params=pltpu.CompilerParams(needs_layout_passes=False),
        scratch_types=[
            pltpu.VMEM((_RPW,), jnp.int32),
            pltpu.VMEM((_RPW, _EMB_DIM), jnp.float32),
            pltpu.SemaphoreType.DMA,
        ],
    )(table, idx)


def kernel(x, embeddings):
    b, c, h, w = x.shape
    fx = jnp.transpose(x, (0, 2, 3, 1)).reshape(b * h * w, c)
    esq = jnp.sum(embeddings ** 2, axis=-1)[:, None]
    fx = pltpu.with_memory_space_constraint(fx, pltpu.MemorySpace.HBM)
    emb_in = pltpu.with_memory_space_constraint(embeddings,
                                                pltpu.MemorySpace.HBM)
    esq = pltpu.with_memory_space_constraint(esq, pltpu.MemorySpace.HBM)
    idx, loss_sum = _vq_argmin(fx, emb_in, esq)
    emb_flat = _vq_gather(embeddings, idx.reshape(-1))
    emb = jnp.transpose(emb_flat.reshape(b, h, w, c), (0, 3, 1, 2))
    loss = loss_sum[0, 0] * ((1.0 + _BETA) / (b * c * h * w))
    return emb, loss


# speedup vs baseline: 3.4373x; 1.5370x over previous
import functools

import jax
import jax.numpy as jnp
from jax.experimental import pallas as pl
from jax.experimental.pallas import tpu as pltpu

_N_EMB = 1024
_EMB_DIM = 256
_BETA = 0.25
_N = 8192
_BLK = 1024


def _vq_body(fx_ref, e_ref, emb_ref, loss_ref):
    fx = fx_ref[...]
    xb = jnp.transpose(fx, (1, 0))
    emb_tab = e_ref[...]
    esq = jnp.sum(emb_tab * emb_tab, axis=1, keepdims=True)

    s = jax.lax.dot_general(
        emb_tab, xb, (((1,), (0,)), ((), ())),
        preferred_element_type=jnp.float32)
    xsq = jnp.sum(xb * xb, axis=0, keepdims=True)
    dist = (xsq + esq) - 2.0 * s

    mind = jnp.min(dist, axis=0, keepdims=True)
    iota = jax.lax.broadcasted_iota(jnp.int32, dist.shape, 0)
    idx = jnp.min(jnp.where(dist == mind, iota, _N_EMB),
                  axis=0, keepdims=True)
    onehot = (iota == idx).astype(jnp.float32)

    gathered = jax.lax.dot_general(
        onehot, emb_tab, (((0,), (0,)), ((), ())),
        preferred_element_type=jnp.float32)
    emb_ref[...] = (gathered + fx) - fx

    @pl.when(pl.program_id(0) == 0)
    def _init():
        loss_ref[...] = jnp.zeros_like(loss_ref)

    loss_ref[...] += jnp.sum(mind, axis=(0, 1), keepdims=True)


@jax.jit
def _vq(fx, embeddings):
    return pl.pallas_call(
        _vq_body,
        grid=(_N // _BLK,),
        in_specs=[
            pl.BlockSpec((_BLK, _EMB_DIM), lambda i: (i, 0)),
            pl.BlockSpec((_N_EMB, _EMB_DIM), lambda i: (0, 0)),
        ],
        out_specs=[
            pl.BlockSpec((_BLK, _EMB_DIM), lambda i: (i, 0)),
            pl.BlockSpec((1, 1), lambda i: (0, 0)),
        ],
        out_shape=[
            jax.ShapeDtypeStruct((_N, _EMB_DIM), jnp.float32),
            jax.ShapeDtypeStruct((1, 1), jnp.float32),
        ],
    )(fx, embeddings)


def kernel(x, embeddings):
    b, c, h, w = x.shape
    fx = jnp.transpose(x, (0, 2, 3, 1)).reshape(b * h * w, c)
    fx = pltpu.with_memory_space_constraint(fx, pltpu.MemorySpace.HBM)
    emb_in = pltpu.with_memory_space_constraint(embeddings,
                                                pltpu.MemorySpace.HBM)
    emb_flat, loss_sum = _vq(fx, emb_in)
    emb = jnp.transpose(emb_flat.reshape(b, h, w, c), (0, 3, 1, 2))
    loss = loss_sum[0, 0] * ((1.0 + _BETA) / (b * c * h * w))
    return emb, loss
